# baseline (speedup 1.0000x reference)
import functools

import jax
import jax.numpy as jnp
from jax.experimental import pallas as pl

_TOP_K = 8
_BM = 512


def _gating_body(x_ref, w_ref, b_ref, o_ref):
    logits = jnp.dot(x_ref[...], w_ref[...], preferred_element_type=jnp.float32)
    logits = logits + b_ref[...]
    cmp = (logits[:, :, None] > logits[:, None, :]).astype(jnp.float32)
    counts = jnp.sum(cmp, axis=1)
    keep = counts < float(_TOP_K)
    m = jnp.max(logits, axis=-1, keepdims=True)
    e = jnp.where(keep, jnp.exp(logits - m), 0.0)
    o_ref[...] = e / jnp.sum(e, axis=-1, keepdims=True)


@jax.jit
def kernel(x, W, b):
    n_tokens, d = x.shape
    n_exp = W.shape[1]
    b2 = b.reshape(1, n_exp)
    grid = (n_tokens // _BM,)
    return pl.pallas_call(
        _gating_body,
        grid=grid,
        in_specs=[
            pl.BlockSpec((_BM, d), lambda i: (i, 0)),
            pl.BlockSpec((d, n_exp), lambda i: (0, 0)),
            pl.BlockSpec((1, n_exp), lambda i: (0, 0)),
        ],
        out_specs=pl.BlockSpec((_BM, n_exp), lambda i: (i, 0)),
        out_shape=jax.ShapeDtypeStruct((n_tokens, n_exp), jnp.float32),
    )(x, W, b2)


# speedup vs baseline: 1.8649x; 1.8649x over previous
import jax
import jax.numpy as jnp
from jax.experimental import pallas as pl
from jax.experimental.pallas import tpu as pltpu

_TOP_K = 8
_BM = 512
_NE = 64


def _xor_partner_rows(x, j):
    if j >= 8:
        n = x.shape[0]
        parts = [x[(b ^ 1) * j:((b ^ 1) * j) + j] for b in range(n // j)]
        return jnp.concatenate(parts, axis=0)
    row = jax.lax.broadcasted_iota(jnp.int32, x.shape, dimension=0)
    lower = (row & j) == 0
    return jnp.where(lower, pltpu.roll(x, x.shape[0] - j, 0), pltpu.roll(x, j, 0))


def _bitonic_desc_rows(x):
    n = x.shape[0]
    row = jax.lax.broadcasted_iota(jnp.int32, x.shape, dimension=0)
    for k_sz in (2, 4, 8, 16, 32, 64):
        j = k_sz // 2
        while j >= 1:
            lower = (row & j) == 0
            partner = _xor_partner_rows(x, j)
            mx = jnp.maximum(x, partner)
            mn = jnp.minimum(x, partner)
            if k_sz < n:
                desc = (row & k_sz) == 0
                take_max = jnp.logical_not(jnp.logical_xor(lower, desc))
            else:
                take_max = lower
            x = jnp.where(take_max, mx, mn)
            j //= 2
    return x


def _gating_body(x_ref, w_ref, b_ref, i512_ref, i64_ref, o_ref):
    logits = jnp.dot(x_ref[...], w_ref[...], preferred_element_type=jnp.float32)
    lt = jax.lax.dot_general(
        logits, i512_ref[...], (((0,), (0,)), ((), ())),
        preferred_element_type=jnp.float32)
    lt = lt + b_ref[...]
    s = _bitonic_desc_rows(lt)
    t = jnp.broadcast_to(s[_TOP_K - 1:_TOP_K, :], lt.shape)
    m = jnp.broadcast_to(s[0:1, :], lt.shape)
    e = jnp.where(lt >= t, jnp.exp(lt - m), 0.0)
    d = e[0:32] + e[32:64]
    d = d[0:16] + d[16:32]
    d = d[0:8] + d[8:16]
    d = d + pltpu.roll(d, 4, 0)
    d = d + pltpu.roll(d, 2, 0)
    d = d + pltpu.roll(d, 1, 0)
    inv = 1.0 / d
    ot = e * jnp.concatenate([inv] * 8, axis=0)
    o_ref[...] = jax.lax.dot_general(
        ot, i64_ref[...], (((0,), (0,)), ((), ())),
        preferred_element_type=jnp.float32)


@jax.jit
def kernel(x, W, b):
    n_tokens, d = x.shape
    n_exp = W.shape[1]
    b2 = b.reshape(n_exp, 1)
    i512 = jnp.eye(_BM, dtype=jnp.float32)
    i64 = jnp.eye(n_exp, dtype=jnp.float32)
    grid = (n_tokens // _BM,)
    return pl.pallas_call(
        _gating_body,
        grid=grid,
        in_specs=[
            pl.BlockSpec((_BM, d), lambda i: (i, 0)),
            pl.BlockSpec((d, n_exp), lambda i: (0, 0)),
            pl.BlockSpec((n_exp, 1), lambda i: (0, 0)),
            pl.BlockSpec((_BM, _BM), lambda i: (0, 0)),
            pl.BlockSpec((n_exp, n_exp), lambda i: (0, 0)),
        ],
        out_specs=pl.BlockSpec((_BM, n_exp), lambda i: (i, 0)),
        out_shape=jax.ShapeDtypeStruct((n_tokens, n_exp), jnp.float32),
    )(x, W, b2, i512, i64)


# speedup vs baseline: 2.0188x; 1.0825x over previous
import jax
import jax.numpy as jnp
from jax.experimental import pallas as pl
from jax.experimental.pallas import tpu as pltpu

_TOP_K = 8
_BM = 512
_NE = 64


def _xor_partner_rows(x, j):
    if j >= 8:
        n = x.shape[0]
        parts = [x[(b ^ 1) * j:((b ^ 1) * j) + j] for b in range(n // j)]
        return jnp.concatenate(parts, axis=0)
    row = jax.lax.broadcasted_iota(jnp.int32, x.shape, dimension=0)
    lower = (row & j) == 0
    return jnp.where(lower, pltpu.roll(x, x.shape[0] - j, 0), pltpu.roll(x, j, 0))


def _bitonic_desc_rows(x):
    n = x.shape[0]
    row = jax.lax.broadcasted_iota(jnp.int32, x.shape, dimension=0)
    for k_sz in (2, 4, 8, 16, 32, 64):
        j = k_sz // 2
        while j >= 1:
            lower = (row & j) == 0
            partner = _xor_partner_rows(x, j)
            mx = jnp.maximum(x, partner)
            mn = jnp.minimum(x, partner)
            if k_sz < n:
                desc = (row & k_sz) == 0
                take_max = jnp.logical_not(jnp.logical_xor(lower, desc))
            else:
                take_max = lower
            x = jnp.where(take_max, mx, mn)
            j //= 2
    return x


def _gating_body(x_ref, w_ref, b_ref, o_ref):
    logits = jnp.dot(x_ref[...], w_ref[...], preferred_element_type=jnp.float32)
    lt = jnp.transpose(logits)
    lt = lt + b_ref[...]
    s = _bitonic_desc_rows(lt)
    t = jnp.broadcast_to(s[_TOP_K - 1:_TOP_K, :], lt.shape)
    m = jnp.broadcast_to(s[0:1, :], lt.shape)
    e = jnp.where(lt >= t, jnp.exp(lt - m), 0.0)
    d = e[0:32] + e[32:64]
    d = d[0:16] + d[16:32]
    d = d[0:8] + d[8:16]
    d = d + pltpu.roll(d, 4, 0)
    d = d + pltpu.roll(d, 2, 0)
    d = d + pltpu.roll(d, 1, 0)
    inv = 1.0 / d
    ot = e * jnp.concatenate([inv] * 8, axis=0)
    o_ref[...] = jnp.transpose(ot)


@jax.jit
def kernel(x, W, b):
    n_tokens, d = x.shape
    n_exp = W.shape[1]
    b2 = b.reshape(n_exp, 1)
    grid = (n_tokens // _BM,)
    return pl.pallas_call(
        _gating_body,
        grid=grid,
        in_specs=[
            pl.BlockSpec((_BM, d), lambda i: (i, 0)),
            pl.BlockSpec((d, n_exp), lambda i: (0, 0)),
            pl.BlockSpec((n_exp, 1), lambda i: (0, 0)),
        ],
        out_specs=pl.BlockSpec((_BM, n_exp), lambda i: (i, 0)),
        out_shape=jax.ShapeDtypeStruct((n_tokens, n_exp), jnp.float32),
    )(x, W, b2)


# speedup vs baseline: 2.1708x; 1.0753x over previous
import jax
import jax.numpy as jnp
from jax.experimental import pallas as pl
from jax.experimental.pallas import tpu as pltpu

_TOP_K = 8
_BM = 1024
_NE = 64


def _xor_partner_rows(x, j):
    if j >= 8:
        n = x.shape[0]
        parts = [x[(b ^ 1) * j:((b ^ 1) * j) + j] for b in range(n // j)]
        return jnp.concatenate(parts, axis=0)
    row = jax.lax.broadcasted_iota(jnp.int32, x.shape, dimension=0)
    lower = (row & j) == 0
    return jnp.where(lower, pltpu.roll(x, x.shape[0] - j, 0), pltpu.roll(x, j, 0))


def _bitonic_desc_rows(x):
    n = x.shape[0]
    row = jax.lax.broadcasted_iota(jnp.int32, x.shape, dimension=0)
    for k_sz in (2, 4, 8, 16, 32, 64):
        j = k_sz // 2
        while j >= 1:
            lower = (row & j) == 0
            partner = _xor_partner_rows(x, j)
            mx = jnp.maximum(x, partner)
            mn = jnp.minimum(x, partner)
            if k_sz < n:
                desc = (row & k_sz) == 0
                take_max = jnp.logical_not(jnp.logical_xor(lower, desc))
            else:
                take_max = lower
            x = jnp.where(take_max, mx, mn)
            j //= 2
    return x


def _gating_body(x_ref, w_ref, b_ref, o_ref):
    logits = jnp.dot(x_ref[...], w_ref[...], preferred_element_type=jnp.float32)
    lt = jnp.transpose(logits)
    lt = lt + b_ref[...]
    s = _bitonic_desc_rows(lt)
    t = jnp.broadcast_to(s[_TOP_K - 1:_TOP_K, :], lt.shape)
    m = jnp.broadcast_to(s[0:1, :], lt.shape)
    e = jnp.where(lt >= t, jnp.exp(lt - m), 0.0)
    d = e[0:32] + e[32:64]
    d = d[0:16] + d[16:32]
    d = d[0:8] + d[8:16]
    d = d + pltpu.roll(d, 4, 0)
    d = d + pltpu.roll(d, 2, 0)
    d = d + pltpu.roll(d, 1, 0)
    inv = 1.0 / d
    ot = e * jnp.concatenate([inv] * 8, axis=0)
    o_ref[...] = jnp.transpose(ot)


@jax.jit
def kernel(x, W, b):
    n_tokens, d = x.shape
    n_exp = W.shape[1]
    b2 = b.reshape(n_exp, 1)
    grid = (n_tokens // _BM,)
    return pl.pallas_call(
        _gating_body,
        grid=grid,
        in_specs=[
            pl.BlockSpec((_BM, d), lambda i: (i, 0)),
            pl.BlockSpec((d, n_exp), lambda i: (0, 0)),
            pl.BlockSpec((n_exp, 1), lambda i: (0, 0)),
        ],
        out_specs=pl.BlockSpec((_BM, n_exp), lambda i: (i, 0)),
        out_shape=jax.ShapeDtypeStruct((n_tokens, n_exp), jnp.float32),
    )(x, W, b2)
